# baseline (speedup 1.0000x reference)
import jax
import jax.numpy as jnp
from jax import lax
from jax.experimental import pallas as pl
from jax.experimental.pallas import tpu as pltpu
from jax.experimental.pallas import tpu_sc as plsc

NC = 2
NS = 16
NW = NC * NS


def _sc_embed(ctx5, cbp, input_ids, tok_sub, pos_table, B, C, D, S):
    L = S + C
    NCH = (L + 15) // 16
    NSLAB = NW // 2
    BW = B // NSLAB
    ND = D // 128
    NH = ND // 2
    CT = C // 8
    DH = NH * 128

    mesh = plsc.VectorSubcoreMesh(core_axis_name="c", subcore_axis_name="s")

    @pl.kernel(
        out_type=jax.ShapeDtypeStruct((L, B, D), jnp.float32),
        mesh=mesh,
        scratch_types=[
            pltpu.VMEM((2, S), jnp.int32),
            pltpu.VMEM((NH, L), jnp.int32),
            pltpu.VMEM((2, NH, L, 128), jnp.float32),
            pltpu.VMEM((L, DH), jnp.float32),
            pltpu.VMEM((CT, NH, 8, 128), jnp.float32),
            pltpu.VMEM((L, DH), jnp.float32),
            pltpu.VMEM((BW,), jnp.int32),
            pltpu.SemaphoreType.DMA,
            pltpu.SemaphoreType.DMA,
            pltpu.SemaphoreType.DMA,
            pltpu.SemaphoreType.DMA,
            pltpu.SemaphoreType.DMA,
            pltpu.SemaphoreType.DMA,
            pltpu.SemaphoreType.DMA,
        ],
        compiler_params=pltpu.CompilerParams(
            needs_layout_passes=False, use_tc_tiling_on_sc=False),
    )
    def k(ctx_hbm, cbp_hbm, ids_hbm, tok_hbm, pos_hbm, out_hbm,
          ids_v, gidx_v, rows_v, wbuf_v, ctx_v, pos_v, cbp_v,
          sem_i0, sem_i1, sem_g0, sem_g1, sem_c, sem_w0, sem_w1):
        wid = lax.axis_index("s") * NC + lax.axis_index("c")
        slab = wid // 2
        half = wid % 2
        dt0 = half * NH
        base = slab * BW
        sem_i = [sem_i0, sem_i1]
        sem_g = [sem_g0, sem_g1]
        sem_w = [sem_w0, sem_w1]

        pltpu.sync_copy(pos_hbm.at[:, pl.ds(dt0 * 128, DH)], pos_v)
        pltpu.sync_copy(cbp_hbm.at[pl.ds(base, BW)], cbp_v)

        def cbp_of(i):
            return plsc.load_gather(cbp_v, [jnp.full((16,), i, jnp.int32)])[0]

        def build_gidx(p, my_cbp):
            for ch in range(NCH):
                j = lax.iota(jnp.int32, 16) + ch * 16
                tj = jnp.where(j < my_cbp, j, j - C)
                tj = jnp.clip(tj, 0, S - 1)
                t = plsc.load_gather(ids_v.at[p], [tj])
                sub = (t >> 3) * (ND * 8) + (t & 7)
                for d in range(NH):
                    if (ch + 1) * 16 <= L:
                        gidx_v[d, pl.ds(ch * 16, 16)] = sub + (dt0 + d) * 8
                    else:
                        plsc.store_scatter(gidx_v.at[d], [j],
                                           sub + (dt0 + d) * 8, mask=j < L)

        def fire_ids(p, i):
            pltpu.async_copy(ids_hbm.at[base + i], ids_v.at[p], sem_i[p])

        def wait_ids(p):
            pltpu.make_async_copy(ids_hbm.at[0], ids_v.at[p], sem_i[p]).wait()

        def fire_g(p):
            for d in range(NH):
                pltpu.async_copy(tok_hbm.at[gidx_v.at[d]],
                                 rows_v.at[p, d], sem_g[p])

        def wait_g(p):
            for d in range(NH):
                pltpu.make_async_copy(tok_hbm.at[gidx_v.at[d]],
                                      rows_v.at[p, d], sem_g[p]).wait()

        def fire_c(i):
            pltpu.async_copy(
                ctx_hbm.at[base + i, :, pl.ds(dt0, NH)], ctx_v, sem_c)

        def wait_c():
            pltpu.make_async_copy(
                ctx_hbm.at[0, :, pl.ds(dt0, NH)], ctx_v, sem_c).wait()

        def fire_w(p, i):
            pltpu.async_copy(wbuf_v,
                             out_hbm.at[:, base + i, pl.ds(dt0 * 128, DH)],
                             sem_w[p])

        def wait_w(p):
            pltpu.make_async_copy(wbuf_v,
                                  out_hbm.at[:, 0, pl.ds(dt0 * 128, DH)],
                                  sem_w[p]).wait()

        def add_pass(p, my_cbp):
            def addrow(r, c2):
                for d in range(NH):
                    for kk in range(8):
                        wsl = pl.ds(d * 128 + kk * 16, 16)
                        wbuf_v[r, wsl] = (rows_v[p, d, r, pl.ds(kk * 16, 16)]
                                          + pos_v[r, wsl])
                return c2
            lax.fori_loop(0, L, addrow, 0, unroll=4)

            def ctxrow(jj, c2):
                r = my_cbp + jj
                for d in range(NH):
                    for kk in range(8):
                        wsl = pl.ds(d * 128 + kk * 16, 16)
                        wbuf_v[r, wsl] = (
                            ctx_v[jj // 8, d, jj % 8, pl.ds(kk * 16, 16)]
                            + pos_v[r, wsl])
                return c2
            lax.fori_loop(0, C, ctxrow, 0, unroll=False)

        fire_ids(0, 0)
        wait_ids(0)
        build_gidx(0, cbp_of(0))
        fire_g(0)
        fire_c(0)
        fire_ids(1, 1)
        last = BW // 2 - 1

        def body(h, carry):
            i = 2 * h
            wait_g(0)
            wait_ids(1)
            build_gidx(1, cbp_of(i + 1))
            fire_g(1)

            @pl.when(h < last)
            def _():
                fire_ids(0, i + 2)

            @pl.when(h > 0)
            def _():
                wait_w(1)
            wait_c()
            add_pass(0, cbp_of(i))
            fire_w(0, i)
            fire_c(i + 1)

            wait_g(1)

            @pl.when(h < last)
            def _():
                wait_ids(0)
                build_gidx(0, cbp_of(i + 2))
                fire_g(0)
                fire_ids(1, i + 3)
            wait_w(0)
            wait_c()
            add_pass(1, cbp_of(i + 1))
            fire_w(1, i + 1)

            @pl.when(h < last)
            def _():
                fire_c(i + 2)
            return carry

        lax.fori_loop(0, BW // 2, body, 0, unroll=False)
        wait_w(1)

    return k(ctx5, cbp, input_ids, tok_sub, pos_table)


def kernel(ctx_embeddings, ctx_begin_pos, input_ids, token_table, pos_table):
    B, C, D = ctx_embeddings.shape
    V, _ = token_table.shape
    _, S = input_ids.shape
    L = S + C
    ND = D // 128
    cbp = ctx_begin_pos.astype(jnp.int32)
    tok_sub = jnp.transpose(
        jnp.reshape(token_table, (V // 8, 8, ND, 128)), (0, 2, 1, 3)
    ).reshape(V // 8 * ND * 8, 128)
    ctx5 = jnp.transpose(
        jnp.reshape(ctx_embeddings, (B, C // 8, 8, ND, 128)), (0, 1, 3, 2, 4))
    emb5 = _sc_embed(ctx5, cbp, input_ids.astype(jnp.int32), tok_sub,
                     pos_table, B, C, D, S)
    emb = jnp.transpose(emb5, (1, 0, 2))
    neg = jnp.finfo(emb.dtype).min
    r = lax.broadcasted_iota(jnp.int32, (L, L), 0)
    c = lax.broadcasted_iota(jnp.int32, (L, L), 1)
    m = jnp.where(c > r, neg, jnp.zeros((), emb.dtype))
    mask = jnp.broadcast_to(m[None, None], (B, 1, L, L))
    return emb, mask


# speedup vs baseline: 1.9261x; 1.9261x over previous
import jax
import jax.numpy as jnp
from jax import lax
from jax.experimental import pallas as pl
from jax.experimental.pallas import tpu as pltpu
from jax.experimental.pallas import tpu_sc as plsc

NC = 2
NS = 16
NW = NC * NS


def _sc_embed(ctx5, cbp, input_ids, tok_sub, pos_table, B, C, D, S):
    L = S + C
    LP = (L + 15) // 16 * 16
    NSLAB = NW // 2
    BW = B // NSLAB
    ND = D // 128
    NH = ND // 2
    CT = C // 8

    mesh = plsc.VectorSubcoreMesh(core_axis_name="c", subcore_axis_name="s")

    @pl.kernel(
        out_type=jax.ShapeDtypeStruct((L, B // 8, ND, 8, 128), jnp.float32),
        mesh=mesh,
        scratch_types=[
            pltpu.VMEM((2, S), jnp.int32),
            pltpu.VMEM((2, NH, LP), jnp.int32),
            pltpu.VMEM((2, NH, LP, 128), jnp.float32),
            pltpu.VMEM((2, CT, NH, 8, 128), jnp.float32),
            pltpu.VMEM((NH, LP, 128), jnp.float32),
            pltpu.VMEM((BW + 16,), jnp.int32),
            pltpu.SemaphoreType.DMA,
            pltpu.SemaphoreType.DMA,
            pltpu.SemaphoreType.DMA,
            pltpu.SemaphoreType.DMA,
            pltpu.SemaphoreType.DMA,
            pltpu.SemaphoreType.DMA,
            pltpu.SemaphoreType.DMA,
            pltpu.SemaphoreType.DMA,
        ],
        compiler_params=pltpu.CompilerParams(
            needs_layout_passes=False, use_tc_tiling_on_sc=False),
    )
    def k(ctx_hbm, cbp_hbm, ids_hbm, tok_hbm, pos_hbm, out_hbm,
          ids_v, gidx_v, rows_v, ctx_v, pos_v, cbp_v,
          sem_i0, sem_i1, sem_g0, sem_g1, sem_c0, sem_c1, sem_w0, sem_w1):
        wid = lax.axis_index("s") * NC + lax.axis_index("c")
        slab = wid // 2
        half = wid % 2
        dt0 = half * NH
        base = slab * BW
        sem_i = [sem_i0, sem_i1]
        sem_g = [sem_g0, sem_g1]
        sem_c = [sem_c0, sem_c1]
        sem_w = [sem_w0, sem_w1]

        for d in range(NH):
            pltpu.sync_copy(pos_hbm.at[:, pl.ds((dt0 + d) * 128, 128)],
                            pos_v.at[d, pl.ds(0, L)])
        pltpu.sync_copy(cbp_hbm.at[pl.ds(base, BW)], cbp_v.at[pl.ds(0, BW)])

        def cbp_of(i):
            return cbp_v[pl.ds(i, 16)][0]

        def build_gidx(p, my_cbp):
            for ch in range(LP // 16):
                j = lax.iota(jnp.int32, 16) + ch * 16
                tj = jnp.where(j < my_cbp, j, j - C)
                tj = jnp.clip(tj, 0, S - 1)
                t = plsc.load_gather(ids_v.at[p], [tj])
                sub = (t >> 3) * (ND * 8) + (t & 7)
                for d in range(NH):
                    gidx_v[p, d, pl.ds(ch * 16, 16)] = sub + (dt0 + d) * 8

        def fire_ids(p, i):
            pltpu.async_copy(ids_hbm.at[base + i], ids_v.at[p], sem_i[p])

        def wait_ids(p):
            pltpu.make_async_copy(ids_hbm.at[0], ids_v.at[p], sem_i[p]).wait()

        def fire_gc(p, i):
            bb = base + i
            for d in range(NH):
                pltpu.async_copy(tok_hbm.at[gidx_v.at[p, d]],
                                 rows_v.at[p, d], sem_g[p])
            pltpu.async_copy(
                ctx_hbm.at[bb, :, pl.ds(dt0, NH)], ctx_v.at[p], sem_c[p])

        def wait_gc(p):
            for d in range(NH):
                pltpu.make_async_copy(tok_hbm.at[gidx_v.at[p, d]],
                                      rows_v.at[p, d], sem_g[p]).wait()
            pltpu.make_async_copy(
                ctx_hbm.at[0, :, pl.ds(dt0, NH)], ctx_v.at[p], sem_c[p]).wait()

        def fire_w(p, i):
            bb = base + i
            bt = bb // 8
            bs = bb % 8
            for d in range(NH):
                pltpu.async_copy(rows_v.at[p, d, pl.ds(0, L)],
                                 out_hbm.at[:, bt, dt0 + d, bs], sem_w[p])

        def wait_w(p):
            for d in range(NH):
                pltpu.make_async_copy(rows_v.at[p, d, pl.ds(0, L)],
                                      out_hbm.at[:, 0, dt0 + d, 0],
                                      sem_w[p]).wait()

        def add_pass(p, my_cbp):
            for d in range(NH):
                def addrow(r, c2, d=d):
                    for kk in range(8):
                        sl = pl.ds(kk * 16, 16)
                        rows_v[p, d, r, sl] = (rows_v[p, d, r, sl]
                                               + pos_v[d, r, sl])
                    return c2
                lax.fori_loop(0, LP, addrow, 0, unroll=4)

            def ctxrow(jj, c2):
                r = my_cbp + jj
                for d in range(NH):
                    for kk in range(8):
                        sl = pl.ds(kk * 16, 16)
                        rows_v[p, d, r, sl] = (
                            ctx_v[p, jj // 8, d, jj % 8, sl] + pos_v[d, r, sl])
                return c2
            lax.fori_loop(0, C, ctxrow, 0, unroll=False)

        fire_ids(0, 0)
        wait_ids(0)
        build_gidx(0, cbp_of(0))
        fire_gc(0, 0)
        fire_ids(1, 1)

        def body(h, carry):
            i = 2 * h
            wait_gc(0)
            wait_ids(1)
            build_gidx(1, cbp_of(i + 1))

            @pl.when(h > 0)
            def _():
                wait_w(1)
            fire_gc(1, i + 1)

            @pl.when(h < BW // 2 - 1)
            def _():
                fire_ids(0, i + 2)
            add_pass(0, cbp_of(i))
            fire_w(0, i)

            wait_gc(1)

            @pl.when(h < BW // 2 - 1)
            def _():
                wait_ids(0)
                build_gidx(0, cbp_of(i + 2))
                wait_w(0)
                fire_gc(0, i + 2)
                fire_ids(1, i + 3)
            add_pass(1, cbp_of(i + 1))
            fire_w(1, i + 1)
            return carry

        lax.fori_loop(0, BW // 2, body, 0, unroll=False)
        wait_w(0)
        wait_w(1)

    return k(ctx5, cbp, input_ids, tok_sub, pos_table)


def kernel(ctx_embeddings, ctx_begin_pos, input_ids, token_table, pos_table):
    B, C, D = ctx_embeddings.shape
    V, _ = token_table.shape
    _, S = input_ids.shape
    L = S + C
    ND = D // 128
    cbp = ctx_begin_pos.astype(jnp.int32)
    tok_sub = jnp.transpose(
        jnp.reshape(token_table, (V // 8, 8, ND, 128)), (0, 2, 1, 3)
    ).reshape(V // 8 * ND * 8, 128)
    ctx5 = jnp.transpose(
        jnp.reshape(ctx_embeddings, (B, C // 8, 8, ND, 128)), (0, 1, 3, 2, 4))
    emb5 = _sc_embed(ctx5, cbp, input_ids.astype(jnp.int32), tok_sub,
                     pos_table, B, C, D, S)
    emb = jnp.transpose(emb5, (1, 3, 0, 2, 4)).reshape(B, L, D)
    neg = jnp.finfo(emb.dtype).min
    r = lax.broadcasted_iota(jnp.int32, (L, L), 0)
    c = lax.broadcasted_iota(jnp.int32, (L, L), 1)
    m = jnp.where(c > r, neg, jnp.zeros((), emb.dtype))
    mask = jnp.broadcast_to(m[None, None], (B, 1, L, L))
    return emb, mask
